# Initial kernel scaffold; baseline (speedup 1.0000x reference)
import jax
import jax.numpy as jnp
from jax.experimental import pallas as pl


def kernel(features, proposals, W1, b1, W2, b2, W_cls, b_cls, W_reg, b_reg):
    raise NotImplementedError("write your pallas kernel here")


# speedup vs baseline: 11.4402x; 11.4402x over previous
import functools

import jax
import jax.numpy as jnp
from jax import lax
from jax.experimental import pallas as pl
from jax.experimental.pallas import tpu as pltpu
from jax.experimental.pallas import tpu_sc as plsc

C, H, W = 256, 200, 304
HW = H * W
RP = 1024
KP = 200
FC = 1024
NCLS = 80
NWORK = 32
BPW = RP // NWORK
KB = 896
NKB = 14


def _prep_body(prop_ref, boxes_ref, idx_ref, wa_ref, wb_ref):
    prop = prop_ref[:, :]
    cx = prop[:, 0:1] * (W - 1.0)
    cy = prop[:, 1:2] * (H - 1.0)
    bw = prop[:, 2:3] * 60.0 + 4.0
    bh = prop[:, 3:4] * 60.0 + 4.0
    x1 = jnp.clip(cx - bw / 2, 0.0, W - 1.0)
    y1 = jnp.clip(cy - bh / 2, 0.0, H - 1.0)
    x2 = jnp.clip(cx + bw / 2, 0.0, W - 1.0)
    y2 = jnp.clip(cy + bh / 2, 0.0, H - 1.0)
    bwe = jnp.maximum(x2 - x1, 1.0)
    bhe = jnp.maximum(y2 - y1, 1.0)
    p49 = jnp.arange(49)
    gx = (((p49 % 7).astype(jnp.float32) + 0.5) / 7.0)[None, :]
    gy = (((p49 // 7).astype(jnp.float32) + 0.5) / 7.0)[None, :]
    xs = x1 + gx * bwe
    ys = y1 + gy * bhe
    x0 = jnp.clip(jnp.floor(xs).astype(jnp.int32), 0, W - 2)
    y0 = jnp.clip(jnp.floor(ys).astype(jnp.int32), 0, H - 2)
    fx = xs - x0.astype(jnp.float32)
    fy = ys - y0.astype(jnp.float32)
    base = y0 * W + x0
    z1 = jnp.zeros((RP, 1), jnp.int32)
    idx_ref[:, :] = jnp.concatenate(
        [base, z1, base + 1, z1, base + W, z1, base + W + 1, z1], axis=1)
    zf = jnp.zeros((RP, 1), jnp.float32)
    wa_ref[:, :] = jnp.concatenate(
        [1 - fy, zf, 1 - fy, zf, fy, zf, fy, zf], axis=1)
    wb_ref[:, :] = jnp.concatenate(
        [1 - fx, zf, fx, zf, 1 - fx, zf, fx, zf], axis=1)
    boxes_ref[:, :] = jnp.concatenate([x1, y1, x2, y2], axis=1)


_prep = pl.pallas_call(
    _prep_body,
    out_shape=(
        jax.ShapeDtypeStruct((RP, 4), jnp.float32),
        jax.ShapeDtypeStruct((RP, KP), jnp.int32),
        jax.ShapeDtypeStruct((RP, KP), jnp.float32),
        jax.ShapeDtypeStruct((RP, KP), jnp.float32),
    ),
)


@functools.partial(
    pl.kernel,
    mesh=plsc.VectorSubcoreMesh(core_axis_name="c", subcore_axis_name="s"),
    out_type=jax.ShapeDtypeStruct((RP, 49 * C), jnp.float32),
    scratch_types=[
        pltpu.VMEM((KP,), jnp.int32),
        pltpu.VMEM((KP, C), jnp.float32),
        pltpu.VMEM((KP * 16,), jnp.float32),
        pltpu.VMEM((KP * 16,), jnp.float32),
        pltpu.VMEM((49 * C,), jnp.float32),
        pltpu.SemaphoreType.DMA,
    ],
)
def _sc_pool(fmap_hbm, idx_hbm, wa_hbm, wb_hbm, out_hbm, idx_v, rows_v,
             wa_v, wb_v, pool_v, sem):
    wid = lax.axis_index("s") * 2 + lax.axis_index("c")

    def per_box(t, carry):
        b = wid * BPW + t
        pltpu.sync_copy(idx_hbm.at[b], idx_v)
        pltpu.sync_copy(wa_hbm.at[b], wa_v)
        pltpu.sync_copy(wb_hbm.at[b], wb_v)
        cp0 = pltpu.async_copy(fmap_hbm.at[idx_v.at[pl.ds(0, 128)]],
                               rows_v.at[pl.ds(0, 128)], sem)
        cp1 = pltpu.async_copy(fmap_hbm.at[idx_v.at[pl.ds(128, 72)]],
                               rows_v.at[pl.ds(128, 72)], sem)
        cp0.wait()
        cp1.wait()

        def per_point(p, c2):
            a0 = wa_v[pl.ds(p * 16, 16)]
            a1 = wa_v[pl.ds((50 + p) * 16, 16)]
            a2 = wa_v[pl.ds((100 + p) * 16, 16)]
            a3 = wa_v[pl.ds((150 + p) * 16, 16)]
            b0 = wb_v[pl.ds(p * 16, 16)]
            b1 = wb_v[pl.ds((50 + p) * 16, 16)]
            b2 = wb_v[pl.ds((100 + p) * 16, 16)]
            b3 = wb_v[pl.ds((150 + p) * 16, 16)]

            def per_chunk(ci, c3):
                s = pl.ds(ci * 16, 16)
                t = (rows_v[p, s] * a0) * b0
                t = t + (rows_v[50 + p, s] * a1) * b1
                t = t + (rows_v[100 + p, s] * a2) * b2
                t = t + (rows_v[150 + p, s] * a3) * b3
                pool_v[pl.ds(p * C + ci * 16, 16)] = t
                return c3

            return lax.fori_loop(0, 16, per_chunk, c2)

        lax.fori_loop(0, 49, per_point, 0)
        pltpu.sync_copy(pool_v, out_hbm.at[b])
        return carry

    lax.fori_loop(0, BPW, per_box, 0)


def _fc1_body(pooled_ref, w1_ref, b1_ref, out_ref):
    k = pl.program_id(0)
    part = jnp.dot(pooled_ref[:, :], w1_ref[:, :],
                   preferred_element_type=jnp.float32)

    @pl.when(k == 0)
    def _():
        out_ref[:, :] = part

    @pl.when(k > 0)
    def _():
        out_ref[:, :] += part

    @pl.when(k == NKB - 1)
    def _():
        out_ref[:, :] = jnp.maximum(out_ref[:, :] + b1_ref[:, :], 0.0)


_fc1 = pl.pallas_call(
    _fc1_body,
    grid=(NKB,),
    in_specs=[
        pl.BlockSpec((RP, KB), lambda k: (0, k)),
        pl.BlockSpec((KB, FC), lambda k: (k, 0)),
        pl.BlockSpec((1, FC), lambda k: (0, 0)),
    ],
    out_specs=pl.BlockSpec((RP, FC), lambda k: (0, 0)),
    out_shape=jax.ShapeDtypeStruct((RP, FC), jnp.float32),
)


def _head_body(h1_ref, w2_ref, b2_ref, wc_ref, bc_ref, wr_ref, br_ref,
               boxes_ref, out_ref, msup_ref):
    h2 = jnp.maximum(
        jnp.dot(h1_ref[:, :], w2_ref[:, :],
                preferred_element_type=jnp.float32) + b2_ref[:, :], 0.0)
    logits = jnp.dot(h2, wc_ref[:, :],
                     preferred_element_type=jnp.float32) + bc_ref[:, :]
    m = jnp.max(logits, axis=1, keepdims=True)
    e = jnp.exp(logits - m)
    zden = jnp.sum(e, axis=1, keepdims=True)
    probs = e / zden
    lane = lax.broadcasted_iota(jnp.int32, (1, 128), 1)
    fg = jnp.where(lane < NCLS, probs, -1.0)
    mfg = jnp.max(fg, axis=1, keepdims=True)
    clsid = jnp.min(jnp.where(fg == mfg, lane, 10000), axis=1, keepdims=True)
    onehot = (lane == clsid).astype(jnp.float32)
    sc = jnp.sum(onehot * probs, axis=1, keepdims=True)
    reg = jnp.dot(h2, wr_ref[:, :],
                  preferred_element_type=jnp.float32) + br_ref[:, :]
    d0 = jnp.sum(onehot * reg[:, 0:128], axis=1, keepdims=True)
    d1 = jnp.sum(onehot * reg[:, 128:256], axis=1, keepdims=True)
    d2 = jnp.sum(onehot * reg[:, 256:384], axis=1, keepdims=True)
    d3 = jnp.sum(onehot * reg[:, 384:512], axis=1, keepdims=True)
    boxes = boxes_ref[:, :]
    widths = boxes[:, 2:3] - boxes[:, 0:1]
    heights = boxes[:, 3:4] - boxes[:, 1:2]
    ctr_x = boxes[:, 0:1] + 0.5 * widths
    ctr_y = boxes[:, 1:2] + 0.5 * heights
    dx = d0 / 10.0
    dy = d1 / 10.0
    dw = jnp.clip(d2 / 5.0, -4.0, 4.0)
    dh = jnp.clip(d3 / 5.0, -4.0, 4.0)
    pcx = dx * widths + ctr_x
    pcy = dy * heights + ctr_y
    pw = jnp.exp(dw) * widths
    ph = jnp.exp(dh) * heights
    px1 = jnp.clip(pcx - 0.5 * pw, 0.0, 303.0)
    py1 = jnp.clip(pcy - 0.5 * ph, 0.0, 303.0)
    px2 = jnp.clip(pcx + 0.5 * pw, 0.0, 303.0)
    py2 = jnp.clip(pcy + 0.5 * ph, 0.0, 303.0)
    rowi = lax.broadcasted_iota(jnp.int32, (RP, 1), 0)
    sc = jnp.where(sc > 0.05, sc, 0.0)
    sc = jnp.where(rowi < 1000, sc, -1.0)

    coli = lax.broadcasted_iota(jnp.int32, (RP, RP), 0)
    rowj = lax.broadcasted_iota(jnp.int32, (RP, RP), 1)

    X = jnp.concatenate(
        [px1, py1, px2, py2, sc, jnp.zeros((RP, 3), jnp.float32)], axis=1)
    XT = jnp.transpose(X)
    sc_row = XT[4:5, :]
    gt = (sc_row > sc).astype(jnp.float32)
    tie = ((sc_row == sc) & (rowj < coli)).astype(jnp.float32)
    rank = jnp.sum(gt + tie, axis=1, keepdims=True)
    rank_row = jnp.transpose(rank)
    ept = (coli.astype(jnp.float32) == rank_row).astype(jnp.float32)
    SX = jnp.dot(ept, X, preferred_element_type=jnp.float32)
    XTe = jnp.transpose(jnp.concatenate([px1, py1, px2, py2], axis=1))
    areas = (px2 - px1) * (py2 - py1)
    xx1 = jnp.maximum(px1, XTe[0:1, :])
    yy1 = jnp.maximum(py1, XTe[1:2, :])
    xx2 = jnp.minimum(px2, XTe[2:3, :])
    yy2 = jnp.minimum(py2, XTe[3:4, :])
    inter = jnp.maximum(xx2 - xx1, 0.0) * jnp.maximum(yy2 - yy1, 0.0)
    areas_row = (XTe[2:3, :] - XTe[0:1, :]) * (XTe[3:4, :] - XTe[1:2, :])
    iou = inter / (areas + areas_row - inter + 1e-9)
    rank_gt = rank_row > rank
    msup_orig = ((iou > 0.5) & rank_gt).astype(jnp.float32)
    er = (rank == rowj.astype(jnp.float32)).astype(jnp.float32)
    msup_ref[:, :] = jnp.dot(
        jnp.dot(ept, msup_orig, preferred_element_type=jnp.float32), er,
        preferred_element_type=jnp.float32)

    rowj1 = lax.broadcasted_iota(jnp.int32, (1, RP), 1)

    def nms_body(i, keep):
        ki = jnp.sum(keep * (rowj1 == i).astype(jnp.float32))
        mrow = msup_ref[pl.ds(i, 1), :]
        return keep * (1.0 - ki * mrow)

    keep = lax.fori_loop(0, 1000, nms_body, jnp.ones((1, RP), jnp.float32))

    realpos = (rowj1 < 1000).astype(jnp.float32)
    k1 = keep * realpos
    k2 = (1.0 - keep) * realpos
    upper = (coli < rowj).astype(jnp.float32)
    pfx1 = jnp.dot(k1, upper, preferred_element_type=jnp.float32)
    pfx2 = jnp.dot(k2, upper, preferred_element_type=jnp.float32)
    n1 = jnp.sum(k1)
    slot = jnp.where(k1 > 0, pfx1, jnp.where(k2 > 0, n1 + pfx2, 1e9))
    t128 = lax.broadcasted_iota(jnp.int32, (128, 1), 0).astype(jnp.float32)
    ssel = (t128 == slot).astype(jnp.float32)
    out_ref[:, :] = jnp.dot(ssel, SX, preferred_element_type=jnp.float32)


_head = pl.pallas_call(
    _head_body,
    out_shape=jax.ShapeDtypeStruct((128, 8), jnp.float32),
    scratch_shapes=[pltpu.VMEM((RP, RP), jnp.float32)],
)


def kernel(features, proposals, W1, b1, W2, b2, W_cls, b_cls, W_reg, b_reg):
    prop = jnp.pad(proposals, ((0, RP - proposals.shape[0]), (0, 0)))
    fmapT = features.reshape(C, HW).T
    boxes, idx, wa, wb = _prep(prop)
    wae = jnp.broadcast_to(wa[:, :, None], (RP, KP, 16)).reshape(RP, KP * 16)
    wbe = jnp.broadcast_to(wb[:, :, None], (RP, KP, 16)).reshape(RP, KP * 16)
    pooled = _sc_pool(fmapT, idx, wae, wbe)
    h1 = _fc1(pooled, W1, b1.reshape(1, FC))
    Wc = jnp.pad(W_cls, ((0, 0), (0, 128 - 81)))
    bc = jnp.pad(b_cls, (0, 128 - 81), constant_values=-1e30).reshape(1, 128)
    Wr = jnp.pad(W_reg.reshape(FC, NCLS, 4).transpose(0, 2, 1),
                 ((0, 0), (0, 0), (0, 48))).reshape(FC, 512)
    br = jnp.pad(b_reg.reshape(NCLS, 4).T, ((0, 0), (0, 48))).reshape(1, 512)
    out = _head(h1, W2, b2.reshape(1, FC), Wc, bc, Wr, br, boxes)
    return out[:100, :5]
